# baseline (speedup 1.0000x reference)
import functools

import jax
import jax.numpy as jnp
from jax import lax
from jax.experimental import pallas as pl
from jax.experimental.pallas import tpu as pltpu
from jax.experimental.pallas import tpu_sc as plsc

EMB = 128
BATCH = 16384
ROWS = (3, 6, 40, 32, 32)
NC, NS = 2, 16
NW = NC * NS
BPW = BATCH // NW
GROUPS = BPW // 16
BLOCK = 2048


def _tables_body(se, sce, oe, xe, ye, w1, b1, t0, t1, t2, t3, t4):
    for i, (src, dst) in enumerate(zip((se, sce, oe, xe, ye),
                                       (t0, t1, t2, t3, t4))):
        acc = lax.dot_general(src[...], w1[pl.ds(i * EMB, EMB), :],
                              (((1,), (0,)), ((), ())),
                              preferred_element_type=jnp.float32)
        dst[...] = acc + (b1[...] * 0.2)[None, :]


def _make_tables(se, sce, oe, xe, ye, w1, b1):
    full = lambda shape: pl.BlockSpec(shape, lambda: (0,) * len(shape))
    return pl.pallas_call(
        _tables_body,
        in_specs=[full((3, EMB)), full((6, EMB)), full((40, EMB)),
                  full((32, EMB)), full((32, EMB)),
                  full((EMB * 5, EMB)), full((EMB,))],
        out_specs=[full((r, EMB)) for r in ROWS],
        out_shape=[jax.ShapeDtypeStruct((r, EMB), jnp.float32) for r in ROWS],
    )(se, sce, oe, xe, ye, w1, b1)


def _gather_body(lat_hbm, t0_hbm, t1_hbm, t2_hbm, t3_hbm, t4_hbm, g_hbm,
                 lat_v, t0_v, t1_v, t2_v, t3_v, t4_v, out_v):
    wid = lax.axis_index("s") * NC + lax.axis_index("c")
    base = wid * BPW
    pltpu.sync_copy(lat_hbm.at[pl.ds(base * 6, BPW * 6)], lat_v)
    pltpu.sync_copy(t0_hbm, t0_v)
    pltpu.sync_copy(t1_hbm, t1_v)
    pltpu.sync_copy(t2_hbm, t2_v)
    pltpu.sync_copy(t3_hbm, t3_v)
    pltpu.sync_copy(t4_hbm, t4_v)
    lane = jnp.arange(16, dtype=jnp.int32)

    def group(gi, carry):
        sids = gi * 16 + lane
        flat = sids * 6
        r0 = plsc.load_gather(lat_v, [flat + 1])
        r1 = plsc.load_gather(lat_v, [flat + 2])
        r2 = plsc.load_gather(lat_v, [flat + 3])
        r3 = plsc.load_gather(lat_v, [flat + 4])
        r4 = plsc.load_gather(lat_v, [flat + 5])
        for j in range(EMB):
            js = jnp.full((16,), j, jnp.int32)
            acc = plsc.load_gather(t0_v, [r0, js])
            acc = acc + plsc.load_gather(t1_v, [r1, js])
            acc = acc + plsc.load_gather(t2_v, [r2, js])
            acc = acc + plsc.load_gather(t3_v, [r3, js])
            acc = acc + plsc.load_gather(t4_v, [r4, js])
            plsc.store_scatter(out_v, [sids, js], acc)
        return carry

    lax.fori_loop(0, GROUPS, group, 0)
    pltpu.sync_copy(out_v, g_hbm.at[pl.ds(base, BPW), :])


@functools.lru_cache(maxsize=1)
def _gather_sum_fn():
    return pl.kernel(
        _gather_body,
        out_type=jax.ShapeDtypeStruct((BATCH, EMB), jnp.float32),
        mesh=plsc.VectorSubcoreMesh(core_axis_name="c", subcore_axis_name="s",
                                    num_cores=NC, num_subcores=NS),
        compiler_params=pltpu.CompilerParams(needs_layout_passes=False),
        scratch_types=[
            pltpu.VMEM((BPW * 6,), jnp.int32),
            pltpu.VMEM((3, EMB), jnp.float32),
            pltpu.VMEM((6, EMB), jnp.float32),
            pltpu.VMEM((40, EMB), jnp.float32),
            pltpu.VMEM((32, EMB), jnp.float32),
            pltpu.VMEM((32, EMB), jnp.float32),
            pltpu.VMEM((BPW, EMB), jnp.float32),
        ])


def _mlp_body(g_ref, w2_ref, b2_ref, out_ref):
    g = g_ref[...]
    h = g * jax.nn.sigmoid(g)
    o = lax.dot_general(h, w2_ref[...], (((1,), (0,)), ((), ())),
                        preferred_element_type=jnp.float32)
    out_ref[...] = o + b2_ref[...][None, :]


def _mlp(g, w2, b2):
    return pl.pallas_call(
        _mlp_body,
        grid=(BATCH // BLOCK,),
        in_specs=[pl.BlockSpec((BLOCK, EMB), lambda i: (i, 0)),
                  pl.BlockSpec((EMB, EMB), lambda i: (0, 0)),
                  pl.BlockSpec((EMB,), lambda i: (0,))],
        out_specs=pl.BlockSpec((BLOCK, EMB), lambda i: (i, 0)),
        out_shape=jax.ShapeDtypeStruct((BATCH, EMB), jnp.float32),
    )(g, w2, b2)


@jax.jit
def kernel(latents, shape_emb, scale_emb, orient_emb, pos_x_emb, pos_y_emb,
           W1, b1, W2, b2):
    t0, t1, t2, t3, t4 = _make_tables(shape_emb, scale_emb, orient_emb,
                                      pos_x_emb, pos_y_emb, W1, b1)
    g = _gather_sum_fn()(latents.reshape(-1), t0, t1, t2, t3, t4)
    return _mlp(g, W2, b2)
